# Initial kernel scaffold; baseline (speedup 1.0000x reference)
import jax
import jax.numpy as jnp
from jax.experimental import pallas as pl


def kernel(x, edge_index, batch, W_in, b_in, eps, W1, b1, W2, b2, gamma, beta, Wp1, bp1, Wp2, bp2):
    raise NotImplementedError("write your pallas kernel here")


# speedup vs baseline: 1.7384x; 1.7384x over previous
import functools

import jax
import jax.numpy as jnp
import numpy as np
from jax import lax
from jax.experimental import pallas as pl
from jax.experimental.pallas import tpu as pltpu
from jax.experimental.pallas import tpu_sc as plsc

N = 10000
E = 320000
D = 128
H = 128
L = 4
G = 64

NUM_CORES = 2
NUM_SUBCORES = 16
ROWS_PER_TILE = 640
N_PAD = ROWS_PER_TILE * NUM_SUBCORES
GARB = N_PAD - 1
SUB = 40

_SC_SIZES = [10080] * 11 + [9840] * 4 + [9760]
SIZES = _SC_SIZES + _SC_SIZES
STARTS = np.concatenate([[0], np.cumsum(SIZES)[:-1]]).astype(np.int32)
TRIPS = (np.array(SIZES) // SUB).astype(np.int32)

ROW_BLK = 2000
N_BLKS = N // ROW_BLK


def _seg_sum_body(h_hbm, srcs_hbm, dsts_hbm, zeros_hbm, out_hbm,
                  idx_s, idx_dv, rows, flushslot,
                  corr1d, corrid1d, mergev, mergei,
                  acc, slots_v, slots_i, sem):
    c = lax.axis_index("c")
    s = lax.axis_index("s")
    w = c * NUM_SUBCORES + s
    lane = lax.iota(jnp.int32, 16)
    garbv = jnp.full((16,), GARB, jnp.int32)
    RHW = ROWS_PER_TILE * H

    pltpu.sync_copy(zeros_hbm, acc.at[pl.ds(s * RHW, RHW)])

    start = c * (E // 2) + jnp.where(s < 11, s * 10080,
                                     110880 + (s - 11) * 9840)
    trip = jnp.where(s < 11, 10080 // SUB,
                     jnp.where(s < 15, 9840 // SUB, 9760 // SUB))
    plsc.subcore_barrier()

    def edge_body(i, carry):
        cur_d, acc8, nrun, first_id = carry
        nd = idx_dv[pl.ds((i // 16) * 16, 16)][i % 16]
        same = nd == cur_d
        do_flush = jnp.logical_and(jnp.logical_not(same), cur_d >= 0)
        flush_first = jnp.logical_and(do_flush, nrun == 0)
        flush_int = jnp.logical_and(do_flush, nrun > 0)

        @pl.when(flush_first)
        def _():
            for k in range(8):
                corr1d[pl.ds(k * 16, 16)] = acc8[k]

        @pl.when(flush_int)
        def _():
            for k in range(8):
                flushslot[pl.ds(k * 16, 16)] = acc8[k]
            offd = pl.multiple_of(cur_d * H, H)
            pltpu.sync_copy(flushslot, acc.at[pl.ds(offd, H)])

        first_id2 = jnp.where(flush_first, cur_d, first_id)
        nrun2 = nrun + jnp.where(do_flush, 1, 0)
        row = tuple(rows[i, pl.ds(k * 16, 16)] for k in range(8))
        acc8b = tuple(jnp.where(same, acc8[k] + row[k], row[k])
                      for k in range(8))
        return (nd, acc8b, nrun2, first_id2)

    def sub_body(j, carry):
        off = pl.multiple_of(start + j * SUB, 8)
        pltpu.sync_copy(srcs_hbm.at[pl.ds(off, SUB)], idx_s)
        pltpu.sync_copy(dsts_hbm.at[pl.ds(off, SUB)], idx_dv.at[pl.ds(0, SUB)])
        pltpu.async_copy(h_hbm.at[idx_s], rows, sem).wait()
        for i in range(SUB):
            carry = edge_body(i, carry)
        return carry

    zeros8 = tuple(jnp.zeros((16,), jnp.float32) for _ in range(8))
    cur_d, acc8, nrun, first_id = lax.fori_loop(
        0, trip, sub_body, (jnp.int32(-1), zeros8, jnp.int32(0),
                            jnp.int32(GARB)))

    @pl.when(nrun == 0)
    def _():
        for k in range(8):
            corr1d[pl.ds(k * 16, 16)] = acc8[k]
            corr1d[pl.ds(H + k * 16, 16)] = jnp.zeros((16,), jnp.float32)

    @pl.when(nrun > 0)
    def _():
        for k in range(8):
            corr1d[pl.ds(H + k * 16, 16)] = acc8[k]

    first_f = jnp.where(nrun == 0, cur_d, first_id)
    last_f = jnp.where(nrun == 0, GARB, cur_d)
    for k in range(8):
        corrid1d[pl.ds(k * 16, 16)] = jnp.full((16,), first_f, jnp.int32)
        corrid1d[pl.ds(H + k * 16, 16)] = jnp.full((16,), last_f, jnp.int32)
    pltpu.sync_copy(corr1d, slots_v.at[pl.ds(s * 2 * H, 2 * H)])
    pltpu.sync_copy(corrid1d, slots_i.at[pl.ds(s * 2 * H, 2 * H)])
    plsc.subcore_barrier()

    @pl.when(s == 0)
    def _():
        pltpu.sync_copy(slots_v, mergev)
        pltpu.sync_copy(slots_i, mergei)
        cur_id = jnp.int32(-1)
        acc8m = tuple(jnp.zeros((16,), jnp.float32) for _ in range(8))
        for slot in range(32):
            sid = mergei[pl.ds(slot * H, 16)][0]
            vals = tuple(mergev[pl.ds(slot * H + k * 16, 16)]
                         for k in range(8))
            skip = sid == GARB
            same = jnp.logical_and(sid == cur_id, jnp.logical_not(skip))
            flush_now = jnp.logical_and(
                jnp.logical_and(jnp.logical_not(same), jnp.logical_not(skip)),
                cur_id >= 0)
            fid = jnp.where(flush_now, cur_id, GARB)
            for k in range(8):
                flushslot[pl.ds(k * 16, 16)] = acc8m[k]
            offd = pl.multiple_of(fid * H, H)
            pltpu.sync_copy(flushslot, acc.at[pl.ds(offd, H)])
            acc8m = tuple(
                jnp.where(same, acc8m[k] + vals[k],
                          jnp.where(skip, acc8m[k], vals[k]))
                for k in range(8))
            cur_id = jnp.where(skip, cur_id, jnp.where(same, cur_id, sid))
        fid = jnp.where(cur_id >= 0, cur_id, GARB)
        for k in range(8):
            flushslot[pl.ds(k * 16, 16)] = acc8m[k]
        offd = pl.multiple_of(fid * H, H)
        pltpu.sync_copy(flushslot, acc.at[pl.ds(offd, H)])

    plsc.subcore_barrier()
    pltpu.sync_copy(acc.at[pl.ds(s * RHW, RHW)],
                    out_hbm.at[pl.ds(c * N_PAD * H + s * RHW, RHW)])


@functools.cache
def _build_seg_sum():
    return pl.kernel(
        _seg_sum_body,
        mesh=plsc.VectorSubcoreMesh(core_axis_name="c", subcore_axis_name="s"),
        out_type=jax.ShapeDtypeStruct((NUM_CORES * N_PAD * H,), jnp.float32),
        scratch_types=[
            pltpu.VMEM((SUB,), jnp.int32),
            pltpu.VMEM((SUB + 16,), jnp.int32),
            pltpu.VMEM((SUB, H), jnp.float32),
            pltpu.VMEM((H,), jnp.float32),
            pltpu.VMEM((2 * H,), jnp.float32),
            pltpu.VMEM((2 * H,), jnp.int32),
            pltpu.VMEM((32 * H,), jnp.float32),
            pltpu.VMEM((32 * H,), jnp.int32),
            pltpu.VMEM_SHARED((N_PAD * H,), jnp.float32),
            pltpu.VMEM_SHARED((32 * H,), jnp.float32),
            pltpu.VMEM_SHARED((32 * H,), jnp.int32),
            pltpu.SemaphoreType.DMA,
        ],
    )


def _seg_sum(h, srcs, dsts, zeros_tile):
    out = _build_seg_sum()(h, srcs, dsts, zeros_tile)
    return out.reshape(NUM_CORES, N_PAD, H)


def _inproj_body(x_ref, wT_ref, b_ref, o_ref):
    o_ref[...] = jnp.maximum(
        jnp.dot(x_ref[...].astype(jnp.bfloat16),
                wT_ref[...].astype(jnp.bfloat16),
                preferred_element_type=jnp.float32) + b_ref[...], 0.0)


def _inproj(x, wT, b):
    return pl.pallas_call(
        _inproj_body,
        grid=(N_BLKS,),
        in_specs=[
            pl.BlockSpec((ROW_BLK, D), lambda i: (i, 0)),
            pl.BlockSpec((D, H), lambda i: (0, 0)),
            pl.BlockSpec((1, H), lambda i: (0, 0)),
        ],
        out_specs=pl.BlockSpec((ROW_BLK, H), lambda i: (i, 0)),
        out_shape=jax.ShapeDtypeStruct((N, H), jnp.float32),
    )(x, wT, b)


def _mlp_body(epsv_ref, h_ref, a0_ref, a1_ref, w1T_ref, b1_ref, w2T_ref,
              b2_ref, z2_ref):
    z = (1.0 + epsv_ref[0]) * h_ref[...] + (a0_ref[0] + a1_ref[0])
    t = jnp.maximum(
        jnp.dot(z.astype(jnp.bfloat16), w1T_ref[...].astype(jnp.bfloat16),
                preferred_element_type=jnp.float32) + b1_ref[...], 0.0)
    z2 = jnp.dot(t.astype(jnp.bfloat16), w2T_ref[...].astype(jnp.bfloat16),
                 preferred_element_type=jnp.float32) + b2_ref[...]
    z2_ref[...] = z2


def _mlp(epsv, h, agg_pad, w1T, b1, w2T, b2):
    return pl.pallas_call(
        _mlp_body,
        grid=(N_BLKS,),
        in_specs=[
            pl.BlockSpec(memory_space=pltpu.SMEM),
            pl.BlockSpec((ROW_BLK, H), lambda i: (i, 0)),
            pl.BlockSpec((1, ROW_BLK, H), lambda i: (0, i, 0)),
            pl.BlockSpec((1, ROW_BLK, H), lambda i: (1, i, 0)),
            pl.BlockSpec((H, 2 * H), lambda i: (0, 0)),
            pl.BlockSpec((1, 2 * H), lambda i: (0, 0)),
            pl.BlockSpec((2 * H, H), lambda i: (0, 0)),
            pl.BlockSpec((1, H), lambda i: (0, 0)),
        ],
        out_specs=pl.BlockSpec((ROW_BLK, H), lambda i: (i, 0)),
        out_shape=jax.ShapeDtypeStruct((N, H), jnp.float32),
    )(epsv, h, agg_pad, agg_pad, w1T, b1, w2T, b2)


def _bnrelu_body(z2_ref, mu_ref, var_ref, g_ref, be_ref, o_ref):
    o_ref[...] = jnp.maximum(
        (z2_ref[...] - mu_ref[...]) / jnp.sqrt(var_ref[...] + 1e-5)
        * g_ref[...] + be_ref[...], 0.0)


def _bnrelu(z2, mu, var, g, be):
    return pl.pallas_call(
        _bnrelu_body,
        grid=(N_BLKS,),
        in_specs=[
            pl.BlockSpec((ROW_BLK, H), lambda i: (i, 0)),
            pl.BlockSpec((1, H), lambda i: (0, 0)),
            pl.BlockSpec((1, H), lambda i: (0, 0)),
            pl.BlockSpec((1, H), lambda i: (0, 0)),
            pl.BlockSpec((1, H), lambda i: (0, 0)),
        ],
        out_specs=pl.BlockSpec((ROW_BLK, H), lambda i: (i, 0)),
        out_shape=jax.ShapeDtypeStruct((N, H), jnp.float32),
    )(z2, mu.reshape(1, H), var.reshape(1, H), g, be)


def _pool_body(h_ref, b_ref, wp1T_ref, bp1_ref, wp2T_ref, bp2_ref, o_ref,
               acc_ref, cnt_ref):
    i = pl.program_id(0)

    @pl.when(i == 0)
    def _():
        acc_ref[...] = jnp.zeros_like(acc_ref)
        cnt_ref[...] = jnp.zeros_like(cnt_ref)

    seg = b_ref[0, 0, :]
    gid = lax.broadcasted_iota(jnp.int32, (G, ROW_BLK), 0)
    mask = jnp.where(seg[None, :] == gid, 1.0, 0.0)
    acc_ref[...] += jnp.dot(mask, h_ref[...],
                            preferred_element_type=jnp.float32,
                            precision=lax.Precision.HIGHEST)
    cnt_ref[...] += jnp.broadcast_to(
        jnp.sum(mask, axis=1, keepdims=True), (G, H))

    @pl.when(i == N_BLKS - 1)
    def _():
        pooled = acc_ref[...] / jnp.maximum(cnt_ref[...], 1.0)
        p = jnp.maximum(
            jnp.dot(pooled.astype(jnp.bfloat16),
                    wp1T_ref[...].astype(jnp.bfloat16),
                    preferred_element_type=jnp.float32) + bp1_ref[...], 0.0)
        o_ref[...] = (
            jnp.dot(p.astype(jnp.bfloat16), wp2T_ref[...].astype(jnp.bfloat16),
                    preferred_element_type=jnp.float32) + bp2_ref[...])


def _pool_pred(h, batch3, wp1T, bp1, wp2T, bp2):
    return pl.pallas_call(
        _pool_body,
        grid=(N_BLKS,),
        in_specs=[
            pl.BlockSpec((ROW_BLK, H), lambda i: (i, 0)),
            pl.BlockSpec((1, 1, ROW_BLK), lambda i: (i, 0, 0)),
            pl.BlockSpec((H, H // 2), lambda i: (0, 0)),
            pl.BlockSpec((1, H // 2), lambda i: (0, 0)),
            pl.BlockSpec((H // 2, 1), lambda i: (0, 0)),
            pl.BlockSpec((1, 1), lambda i: (0, 0)),
        ],
        out_specs=pl.BlockSpec((G, 1), lambda i: (0, 0)),
        out_shape=jax.ShapeDtypeStruct((G, 1), jnp.float32),
        scratch_shapes=[
            pltpu.VMEM((G, H), jnp.float32),
            pltpu.VMEM((G, H), jnp.float32),
        ],
    )(h, batch3, wp1T, bp1, wp2T, bp2)


def kernel(x, edge_index, batch, W_in, b_in, eps, W1, b1, W2, b2, gamma, beta,
           Wp1, bp1, Wp2, bp2):
    src = edge_index[0]
    dst = edge_index[1]
    perm = jnp.argsort(dst, stable=True)
    srcs = jnp.take(src, perm)
    dsts = jnp.take(dst, perm)
    zeros_tile = jnp.zeros((ROWS_PER_TILE * H,), jnp.float32)

    h = _inproj(x, W_in.T, b_in.reshape(1, H))
    W1T = jnp.transpose(W1, (0, 2, 1))
    W2T = jnp.transpose(W2, (0, 2, 1))
    for l in range(L):
        agg = _seg_sum(h, srcs, dsts, zeros_tile)
        z2 = _mlp(eps[l].reshape(1), h, agg,
                  W1T[l], b1[l].reshape(1, 2 * H),
                  W2T[l], b2[l].reshape(1, H))
        mu = jnp.mean(z2, axis=0)
        var = jnp.var(z2, axis=0)
        h = _bnrelu(z2, mu, var, gamma[l].reshape(1, H),
                    beta[l].reshape(1, H))

    return _pool_pred(h, batch.reshape(N_BLKS, 1, ROW_BLK),
                      Wp1.T, bp1.reshape(1, H // 2),
                      Wp2.T, bp2.reshape(1, 1))
